# Initial kernel scaffold; baseline (speedup 1.0000x reference)
import jax
import jax.numpy as jnp
from jax.experimental import pallas as pl


def kernel(wav):
    raise NotImplementedError("write your pallas kernel here")


# speedup vs baseline: 3.5165x; 3.5165x over previous
import functools

import jax
import jax.numpy as jnp
from jax import lax
from jax.experimental import pallas as pl
from jax.experimental.pallas import tpu as pltpu
from jax.experimental.pallas import tpu_sc as plsc

_SHIFT = 8192
_NUM_CORES = 2
_NUM_SUBCORES = 16


def _shift_gather(wav2, offs, rows, length):
    num_workers = _NUM_CORES * _NUM_SUBCORES
    rows_per_worker = rows // num_workers

    mesh = plsc.VectorSubcoreMesh(
        core_axis_name="core", subcore_axis_name="subcore",
        num_cores=_NUM_CORES, num_subcores=_NUM_SUBCORES,
    )

    chunk = 63488
    num_chunks = length // chunk

    @functools.partial(
        pl.kernel,
        out_type=jax.ShapeDtypeStruct((rows, length), jnp.float32),
        mesh=mesh,
        scratch_types=[
            pltpu.VMEM((num_workers, 16), jnp.int32),
            pltpu.VMEM((chunk + 8,), jnp.float32),
            pltpu.SemaphoreType.DMA,
            pltpu.SemaphoreType.DMA,
        ],
        compiler_params=pltpu.CompilerParams(use_tc_tiling_on_sc=False),
    )
    def shift_copy(wav_hbm, off_hbm, out_hbm, off_v, buf_v, sem_in, sem_out):
        wid = lax.axis_index("subcore") * _NUM_CORES + lax.axis_index("core")
        pltpu.sync_copy(off_hbm, off_v)
        my_offs = off_v[wid]
        for r in range(rows_per_worker):
            row = wid * rows_per_worker + r
            off = my_offs[r]
            base = pl.multiple_of((off // 8) * 8, 8)
            rem = off - base
            for j in range(num_chunks):
                pltpu.async_copy(
                    wav_hbm.at[row, pl.ds(base + j * chunk, chunk + 8)],
                    buf_v,
                    sem_in,
                ).wait()

                @pl.when(rem != 0)
                def _shift():
                    def body(k, _):
                        buf_v[pl.ds(k * 16, 16)] = buf_v[pl.ds(rem + k * 16, 16)]
                        return _

                    lax.fori_loop(0, chunk // 16, body, None)

                pltpu.async_copy(
                    buf_v.at[pl.ds(0, chunk)],
                    out_hbm.at[row, pl.ds(j * chunk, chunk)],
                    sem_out,
                ).wait()

    return shift_copy(wav2, offs)


def kernel(wav):
    sources, batch, channels, length0 = wav.shape
    length = length0 - _SHIFT
    okey = jax.random.fold_in(jax.random.key(0), 1)
    offsets = jax.random.randint(
        okey, (sources, batch, 1, 1), 0, _SHIFT, dtype=jnp.int32
    )
    rows = sources * batch * channels
    num_workers = _NUM_CORES * _NUM_SUBCORES
    offs = jnp.broadcast_to(offsets, (sources, batch, channels, 1)).reshape(
        num_workers, rows // num_workers
    )
    offs = jnp.pad(offs, ((0, 0), (0, 16 - rows // num_workers)))
    wav2 = wav.reshape(rows, length0)
    out2 = _shift_gather(wav2, offs, rows, length)
    return out2.reshape(sources, batch, channels, length)


# speedup vs baseline: 5.5948x; 1.5910x over previous
import functools

import jax
import jax.numpy as jnp
from jax import lax
from jax.experimental import pallas as pl
from jax.experimental.pallas import tpu as pltpu
from jax.experimental.pallas import tpu_sc as plsc

_SHIFT = 8192
_NUM_CORES = 2
_NUM_SUBCORES = 16


def _shift_gather(wav2, offs, rows, length):
    num_workers = _NUM_CORES * _NUM_SUBCORES
    rows_per_worker = rows // num_workers

    chunk = 31744
    num_chunks = length // chunk

    mesh = plsc.VectorSubcoreMesh(
        core_axis_name="core", subcore_axis_name="subcore",
        num_cores=_NUM_CORES, num_subcores=_NUM_SUBCORES,
    )

    @functools.partial(
        pl.kernel,
        out_type=jax.ShapeDtypeStruct((rows, length), jnp.float32),
        mesh=mesh,
        scratch_types=[
            pltpu.VMEM((num_workers, 16), jnp.int32),
            pltpu.VMEM((chunk + 8,), jnp.float32),
            pltpu.VMEM((chunk + 8,), jnp.float32),
            pltpu.VMEM((chunk,), jnp.float32),
            pltpu.VMEM((chunk,), jnp.float32),
            pltpu.SemaphoreType.DMA,
            pltpu.SemaphoreType.DMA,
            pltpu.SemaphoreType.DMA,
            pltpu.SemaphoreType.DMA,
        ],
        compiler_params=pltpu.CompilerParams(use_tc_tiling_on_sc=False),
    )
    def shift_copy(wav_hbm, off_hbm, out_hbm, off_v,
                   ib0, ib1, ob0, ob1, si0, si1, so0, so1):
        ibufs, obufs = (ib0, ib1), (ob0, ob1)
        isems, osems = (si0, si1), (so0, so1)
        wid = lax.axis_index("subcore") * _NUM_CORES + lax.axis_index("core")
        pltpu.sync_copy(off_hbm, off_v)
        my_offs = off_v[wid]

        rows_ = []
        bases = []
        rems = []
        for r in range(rows_per_worker):
            off = my_offs[r]
            rows_.append(wid * rows_per_worker + r)
            bases.append(pl.multiple_of((off // 8) * 8, 8))
            rems.append(off - bases[-1])

        tasks = [(r, j) for r in range(rows_per_worker)
                 for j in range(num_chunks)]
        ntasks = len(tasks)

        def in_desc(tt):
            r, j = tasks[tt]
            return pltpu.make_async_copy(
                wav_hbm.at[rows_[r], pl.ds(bases[r] + j * chunk, chunk + 8)],
                ibufs[tt % 2], isems[tt % 2])

        def out_desc(tt):
            r, j = tasks[tt]
            return pltpu.make_async_copy(
                obufs[tt % 2],
                out_hbm.at[rows_[r], pl.ds(j * chunk, chunk)],
                osems[tt % 2])

        def shift(tt):
            r, _ = tasks[tt]
            rem = rems[r]
            ib, ob = ibufs[tt % 2], obufs[tt % 2]

            @functools.partial(plsc.parallel_loop, 0, chunk // 16, unroll=8)
            def _(k):
                ob[pl.ds(k * 16, 16)] = ib[pl.ds(rem + k * 16, 16)]

        in_desc(0).start()
        for tt in range(ntasks):
            if tt + 1 < ntasks:
                in_desc(tt + 1).start()
            in_desc(tt).wait()
            if tt >= 2:
                out_desc(tt - 2).wait()
            shift(tt)
            out_desc(tt).start()
        out_desc(ntasks - 2).wait()
        out_desc(ntasks - 1).wait()

    return shift_copy(wav2, offs)


def kernel(wav):
    sources, batch, channels, length0 = wav.shape
    length = length0 - _SHIFT
    okey = jax.random.fold_in(jax.random.key(0), 1)
    offsets = jax.random.randint(
        okey, (sources, batch, 1, 1), 0, _SHIFT, dtype=jnp.int32
    )
    rows = sources * batch * channels
    num_workers = _NUM_CORES * _NUM_SUBCORES
    offs = jnp.broadcast_to(offsets, (sources, batch, channels, 1)).reshape(
        num_workers, rows // num_workers
    )
    offs = jnp.pad(offs, ((0, 0), (0, 16 - rows // num_workers)))
    wav2 = wav.reshape(rows, length0)
    out2 = _shift_gather(wav2, offs, rows, length)
    return out2.reshape(sources, batch, channels, length)
